# baseline (speedup 1.0000x reference)
import functools

import numpy as np
import jax
import jax.numpy as jnp
from jax import lax
from jax.experimental import pallas as pl
from jax.experimental.pallas import tpu as pltpu, tpu_sc as plsc

_PROB = 0.25

_DROPPED_B16 = (7, 6, 3, 2)


@functools.lru_cache(maxsize=None)
def _dropped_ids(b):
    num = 1 if b == 1 else int(b * _PROB)
    if b == 16:
        return _DROPPED_B16[:num]
    with jax.ensure_compile_time_eval(), jax.default_device(jax.devices("cpu")[0]):
        perm = np.asarray(jax.random.permutation(jax.random.key(1), b))
    return tuple(int(x) for x in perm[:num])


def kernel(emb0):
    b, t, c, d = emb0.shape
    dropped = set(_dropped_ids(b))

    info = plsc.get_sparse_core_info()
    nw = info.num_cores * info.num_subcores
    wpb = nw // b
    tn = t // wpb
    mesh = plsc.VectorSubcoreMesh(core_axis_name="c", subcore_axis_name="s")

    zeros = jnp.zeros((tn, c, d), emb0.dtype)

    @functools.partial(
        pl.kernel,
        out_type=jax.ShapeDtypeStruct((b, t, c, d), emb0.dtype),
        mesh=mesh,
        scratch_types=[pltpu.SemaphoreType.DMA],
    )
    def run(in_hbm, z_hbm, out_hbm, sem):
        wid = lax.axis_index("s") * info.num_cores + lax.axis_index("c")
        bw = wid // wpb
        h = wid % wpb
        t0 = h * tn
        is_drop = functools.reduce(
            jnp.logical_or, [bw == i for i in dropped], jnp.bool_(False)
        )
        is_last = h == wpb - 1

        @pl.when(jnp.logical_not(is_drop))
        def _copy():
            pltpu.async_copy(
                in_hbm.at[bw, pl.ds(t0, tn)], out_hbm.at[bw, pl.ds(t0, tn)], sem
            ).wait()

        @pl.when(jnp.logical_and(is_drop, jnp.logical_not(is_last)))
        def _zero_full():
            pltpu.async_copy(
                z_hbm.at[pl.ds(0, tn)], out_hbm.at[bw, pl.ds(t0, tn)], sem
            ).wait()

        @pl.when(jnp.logical_and(is_drop, is_last))
        def _zero_tail():
            pltpu.async_copy(
                z_hbm.at[pl.ds(0, tn - 1)], out_hbm.at[bw, pl.ds(t0, tn - 1)], sem
            ).wait()
            pltpu.async_copy(
                in_hbm.at[bw, pl.ds(t - 1, 1)], out_hbm.at[bw, pl.ds(t - 1, 1)], sem
            ).wait()

    return run(emb0, zeros)


# speedup vs baseline: 32.6852x; 32.6852x over previous
import functools

import numpy as np
import jax
import jax.numpy as jnp
from jax import lax
from jax.experimental import pallas as pl
from jax.experimental.pallas import tpu as pltpu, tpu_sc as plsc

_PROB = 0.25

_DROPPED_B16 = (7, 6, 3, 2)


@functools.lru_cache(maxsize=None)
def _dropped_ids(b):
    num = 1 if b == 1 else int(b * _PROB)
    if b == 16:
        return _DROPPED_B16[:num]
    with jax.ensure_compile_time_eval(), jax.default_device(jax.devices("cpu")[0]):
        perm = np.asarray(jax.random.permutation(jax.random.key(1), b))
    return tuple(int(x) for x in perm[:num])


def kernel(emb0):
    b, t, c, d = emb0.shape
    dropped = set(_dropped_ids(b))

    info = plsc.get_sparse_core_info()
    nw = info.num_cores * info.num_subcores
    wpb = nw // b
    tn = t // wpb
    ch = 64
    nch = tn // ch
    mesh = plsc.VectorSubcoreMesh(core_axis_name="c", subcore_axis_name="s")

    zeros = jnp.zeros((ch, c, d), emb0.dtype)

    @functools.partial(
        pl.kernel,
        out_type=jax.ShapeDtypeStruct((b, t, c, d), emb0.dtype),
        mesh=mesh,
        scratch_types=[
            pltpu.VMEM((ch, c, d), emb0.dtype),
            pltpu.VMEM((ch, c, d), emb0.dtype),
            pltpu.SemaphoreType.DMA,
            pltpu.SemaphoreType.DMA,
            pltpu.SemaphoreType.DMA,
            pltpu.SemaphoreType.DMA,
        ],
    )
    def run(in_hbm, z_hbm, out_hbm, buf0, buf1, is0, is1, os0, os1):
        wid = lax.axis_index("s") * info.num_cores + lax.axis_index("c")
        bw = wid // wpb
        h = wid % wpb
        t0 = h * tn
        is_drop = functools.reduce(
            jnp.logical_or, [bw == i for i in dropped], jnp.bool_(False)
        )
        is_last = h == wpb - 1
        bufs = (buf0, buf1)
        isems = (is0, is1)
        osems = (os0, os1)

        def src(i):
            return in_hbm.at[bw, pl.ds(t0 + i * ch, ch)]

        def dst(i):
            return out_hbm.at[bw, pl.ds(t0 + i * ch, ch)]

        @pl.when(jnp.logical_not(is_drop))
        def _copy():
            in_d = [None] * nch
            out_d = [None] * nch
            in_d[0] = pltpu.async_copy(src(0), bufs[0], isems[0])
            for i in range(nch):
                p = i % 2
                if i + 1 < nch:
                    q = (i + 1) % 2
                    if i >= 1:
                        out_d[i - 1].wait()
                    in_d[i + 1] = pltpu.async_copy(src(i + 1), bufs[q], isems[q])
                in_d[i].wait()
                out_d[i] = pltpu.async_copy(bufs[p], dst(i), osems[p])
            out_d[nch - 2].wait()
            out_d[nch - 1].wait()

        @pl.when(is_drop)
        def _zero():
            pltpu.async_copy(z_hbm.at[pl.ds(0, ch)], buf0, is0).wait()
            out_d = [pltpu.async_copy(buf0, dst(i), os0) for i in range(nch - 1)]
            for d_ in out_d:
                d_.wait()

            @pl.when(jnp.logical_not(is_last))
            def _full_tail():
                pltpu.async_copy(buf0, dst(nch - 1), os0).wait()

            @pl.when(is_last)
            def _partial_tail():
                pltpu.async_copy(
                    buf0.at[pl.ds(0, ch - 1)],
                    out_hbm.at[bw, pl.ds(t0 + (nch - 1) * ch, ch - 1)],
                    os0,
                ).wait()
                pltpu.async_copy(
                    in_hbm.at[bw, pl.ds(t - 1, 1)], buf1.at[pl.ds(0, 1)], is1
                ).wait()
                pltpu.async_copy(
                    buf1.at[pl.ds(0, 1)], out_hbm.at[bw, pl.ds(t - 1, 1)], os1
                ).wait()

    return run(emb0, zeros)


# speedup vs baseline: 32.8141x; 1.0039x over previous
import functools

import numpy as np
import jax
import jax.numpy as jnp
from jax import lax
from jax.experimental import pallas as pl
from jax.experimental.pallas import tpu as pltpu, tpu_sc as plsc

_PROB = 0.25

_DROPPED_B16 = (7, 6, 3, 2)


@functools.lru_cache(maxsize=None)
def _dropped_ids(b):
    num = 1 if b == 1 else int(b * _PROB)
    if b == 16:
        return _DROPPED_B16[:num]
    with jax.ensure_compile_time_eval(), jax.default_device(jax.devices("cpu")[0]):
        perm = np.asarray(jax.random.permutation(jax.random.key(1), b))
    return tuple(int(x) for x in perm[:num])


def kernel(emb0):
    b, t, c, d = emb0.shape
    dropped = set(_dropped_ids(b))

    info = plsc.get_sparse_core_info()
    nw = info.num_cores * info.num_subcores
    wpb = nw // b
    tn = t // wpb
    ch = 64
    nch = tn // ch
    mesh = plsc.VectorSubcoreMesh(core_axis_name="c", subcore_axis_name="s")

    zeros = jnp.zeros((ch, c, d), emb0.dtype)

    @functools.partial(
        pl.kernel,
        out_type=jax.ShapeDtypeStruct((b, t, c, d), emb0.dtype),
        mesh=mesh,
        scratch_types=[
            pltpu.VMEM((ch, c, d), emb0.dtype),
            pltpu.VMEM((ch, c, d), emb0.dtype),
            pltpu.VMEM((ch, c, d), emb0.dtype),
            pltpu.SemaphoreType.DMA,
            pltpu.SemaphoreType.DMA,
            pltpu.SemaphoreType.DMA,
            pltpu.SemaphoreType.DMA,
            pltpu.SemaphoreType.DMA,
            pltpu.SemaphoreType.DMA,
        ],
    )
    def run(in_hbm, z_hbm, out_hbm, buf0, buf1, buf2, is0, is1, is2, os0, os1, os2):
        wid = lax.axis_index("s") * info.num_cores + lax.axis_index("c")
        bw = wid // wpb
        h = wid % wpb
        t0 = h * tn
        is_drop = functools.reduce(
            jnp.logical_or, [bw == i for i in dropped], jnp.bool_(False)
        )
        is_last = h == wpb - 1
        bufs = (buf0, buf1, buf2)
        isems = (is0, is1, is2)
        osems = (os0, os1, os2)
        nring = len(bufs)

        def src(i):
            return in_hbm.at[bw, pl.ds(t0 + i * ch, ch)]

        def dst(i):
            return out_hbm.at[bw, pl.ds(t0 + i * ch, ch)]

        @pl.when(jnp.logical_not(is_drop))
        def _copy():
            in_d = [None] * nch
            out_d = [None] * nch
            in_d[0] = pltpu.async_copy(src(0), bufs[0], isems[0])
            for i in range(nch):
                p = i % nring
                if i + 1 < nch:
                    q = (i + 1) % nring
                    if i + 1 >= nring:
                        out_d[i + 1 - nring].wait()
                    in_d[i + 1] = pltpu.async_copy(src(i + 1), bufs[q], isems[q])
                in_d[i].wait()
                out_d[i] = pltpu.async_copy(bufs[p], dst(i), osems[p])
            for j in range(max(0, nch - nring), nch):
                out_d[j].wait()

        @pl.when(is_drop)
        def _zero():
            pltpu.async_copy(z_hbm.at[pl.ds(0, ch)], buf0, is0).wait()
            out_d = [pltpu.async_copy(buf0, dst(i), os0) for i in range(nch - 1)]
            for d_ in out_d:
                d_.wait()

            @pl.when(jnp.logical_not(is_last))
            def _full_tail():
                pltpu.async_copy(buf0, dst(nch - 1), os0).wait()

            @pl.when(is_last)
            def _partial_tail():
                pltpu.async_copy(
                    buf0.at[pl.ds(0, ch - 1)],
                    out_hbm.at[bw, pl.ds(t0 + (nch - 1) * ch, ch - 1)],
                    os0,
                ).wait()
                pltpu.async_copy(
                    in_hbm.at[bw, pl.ds(t - 1, 1)], buf1.at[pl.ds(0, 1)], is1
                ).wait()
                pltpu.async_copy(
                    buf1.at[pl.ds(0, 1)], out_hbm.at[bw, pl.ds(t - 1, 1)], os1
                ).wait()

    return run(emb0, zeros)
